# baseline (speedup 1.0000x reference)
import functools

import jax
import jax.numpy as jnp
from jax import lax
from jax.experimental import pallas as pl
from jax.experimental.pallas import tpu as pltpu
from jax.experimental.pallas import tpu_sc as plsc

B = 16384
D = 64
RB = 2048
NRB = 489
V2 = NRB * (RB // 2)
NC = 2
NS = 16
NW = NC * NS
BPW = B // NW
NCH = 4
CH = BPW // NCH
NPASS = 2
CPP = NCH // NPASS
PCH = BPW // NPASS
REG_W = 1e-4
SREG_W = 1e-3

_mesh = plsc.VectorSubcoreMesh(
    core_axis_name="c", subcore_axis_name="s", num_cores=NC, num_subcores=NS
)


@functools.partial(
    pl.kernel,
    out_type=[
        jax.ShapeDtypeStruct((B,), jnp.float32),
        jax.ShapeDtypeStruct((NW, 16), jnp.float32),
    ],
    mesh=_mesh,
    scratch_types=[
        pltpu.VMEM((NCH, CH), jnp.int32),
        pltpu.VMEM((NCH, CH), jnp.int32),
        pltpu.VMEM((NCH, CH), jnp.int32),
        pltpu.VMEM((NCH, CH), jnp.int32),
        pltpu.VMEM((NCH, CH), jnp.int32),
        pltpu.VMEM((NCH, CH), jnp.int32),
        pltpu.VMEM((PCH, 2 * D), jnp.float32),
        pltpu.VMEM((PCH, 2 * D), jnp.float32),
        pltpu.VMEM((PCH, 2 * D), jnp.float32),
        pltpu.VMEM((BPW,), jnp.float32),
        pltpu.VMEM((16,), jnp.float32),
        pltpu.VMEM((16,), jnp.float32),
        pltpu.SemaphoreType.DMA,
    ],
)
def _sc_gather(u_idx, p_idx, n_idx, thr, t2, diffs_out, regs_out,
               iu, ip_, in_, hu, hp, hn, ru, rp, rn, dbuf, thrv, rstage, sem):
    wid = lax.axis_index("s") * NC + lax.axis_index("c")
    ibase = wid * NCH

    pltpu.sync_copy(u_idx.at[pl.ds(ibase, NCH)], iu)
    pltpu.sync_copy(p_idx.at[pl.ds(ibase, NCH)], ip_)
    pltpu.sync_copy(n_idx.at[pl.ds(ibase, NCH)], in_)
    pltpu.sync_copy(thr, thrv)

    def pack_ids(j, c):
        for src, dst in ((iu, hu), (ip_, hp), (in_, hn)):
            def hbody(g, cc):
                sl = pl.ds(g * 16, 16)
                v = src[j, sl]
                dst[j, sl] = (
                    lax.shift_left(lax.shift_right_logical(v, 11), 10)
                    | (v & 1023)
                )
                return cc
            lax.fori_loop(0, CH // 16, hbody, 0)
        return c

    for j in range(NCH):
        pack_ids(j, 0)

    t = thrv[...]
    nt = -t
    lanes = lax.iota(jnp.int32, 16)

    def lane_sum(x):
        for sft in (8, 4, 2, 1):
            x = x + x.at[lanes ^ sft].get(mode="promise_in_bounds")
        return x

    racc = jnp.zeros((16,), jnp.float32)
    for ps in range(NPASS):
        copies = []
        for jj in range(CPP):
            j = ps * CPP + jj
            dst = pl.ds(jj * CH, CH)
            copies.append(pltpu.async_copy(t2.at[hu.at[j]], ru.at[dst], sem))
            copies.append(pltpu.async_copy(t2.at[hp.at[j]], rp.at[dst], sem))
            copies.append(pltpu.async_copy(t2.at[hn.at[j]], rn.at[dst], sem))
        for c in copies:
            c.wait()

        for jj in range(CPP):
            j = ps * CPP + jj

            def group(g, racc, jj=jj, j=j):
                isl = pl.ds(g * 16, 16)
                ou = iu[j, isl]
                op = ip_[j, isl]
                on = in_[j, isl]
                dv = jnp.zeros((16,), jnp.float32)
                for r in range(16):
                    row = jj * CH + g * 16 + r
                    fu = (lax.shift_right_logical(ou[r], 10) & 1) * D
                    fp = (lax.shift_right_logical(op[r], 10) & 1) * D
                    fn = (lax.shift_right_logical(on[r], 10) & 1) * D
                    acc = jnp.zeros((16,), jnp.float32)
                    for k in range(D // 16):
                        u = ru[row, pl.ds(fu + k * 16, 16)]
                        p = rp[row, pl.ds(fp + k * 16, 16)]
                        n = rn[row, pl.ds(fn + k * 16, 16)]
                        su = u - jnp.minimum(jnp.maximum(u, nt), t)
                        sp = p - jnp.minimum(jnp.maximum(p, nt), t)
                        sn = n - jnp.minimum(jnp.maximum(n, nt), t)
                        acc = acc + su * (sn - sp)
                        racc = racc + (u * u + p * p + n * n)
                    dv = jnp.where(lanes == r, lane_sum(acc), dv)
                dbuf[pl.ds(j * CH + g * 16, 16)] = dv
                return racc

            racc = lax.fori_loop(0, CH // 16, group, racc)

    rstage[...] = racc
    pltpu.sync_copy(dbuf, diffs_out.at[pl.ds(wid * BPW, BPW)])
    pltpu.sync_copy(rstage, regs_out.at[wid])


def _repack_body(x_ref, o_ref):
    x = x_ref[...]
    i0 = lax.broadcasted_iota(jnp.int32, (D, D), 0)
    i1 = lax.broadcasted_iota(jnp.int32, (D, D), 1)
    ident = (i0 == i1).astype(jnp.float32)
    xt = lax.dot_general(x, ident, (((0,), (0,)), ((), ())),
                         preferred_element_type=jnp.float32)
    o_ref[:, 0:D] = xt[0:RB // 2]
    o_ref[:, D:2 * D] = xt[RB // 2:RB]


_repack = pl.pallas_call(
    _repack_body,
    grid=(NRB,),
    in_specs=[pl.BlockSpec((D, RB), lambda b: (0, b))],
    out_specs=pl.BlockSpec((RB // 2, 2 * D), lambda b: (b, 0)),
    out_shape=jax.ShapeDtypeStruct((V2, 2 * D), jnp.float32),
    compiler_params=pltpu.CompilerParams(fuse_transposed_lhs_in_matmul=True),
)


def _tc_body(d_ref, r_ref, s_ref, loss_ref, le_ref, reg_ref, sl_ref):
    diff = d_ref[...]
    le = jnp.mean(jax.nn.softplus(diff))
    reg = 0.5 * jnp.sum(r_ref[...]) * (1.0 / B) * REG_W
    sv = s_ref[0]
    sl = 0.5 * sv * sv * (1.0 / B) * SREG_W
    le_ref[0] = le
    reg_ref[0] = reg
    sl_ref[0] = sl
    loss_ref[0] = le + reg + sl


_tc_final = pl.pallas_call(
    _tc_body,
    out_shape=[jax.ShapeDtypeStruct((1,), jnp.float32)] * 4,
    in_specs=[
        pl.BlockSpec(memory_space=pltpu.VMEM),
        pl.BlockSpec(memory_space=pltpu.VMEM),
        pl.BlockSpec(memory_space=pltpu.SMEM),
    ],
    out_specs=[pl.BlockSpec(memory_space=pltpu.SMEM)] * 4,
)


def kernel(users, pos, neg, table, s):
    s = s.astype(jnp.float32)
    u2 = users.astype(jnp.int32).reshape(B // CH, CH)
    p2 = pos.astype(jnp.int32).reshape(B // CH, CH)
    n2 = neg.astype(jnp.int32).reshape(B // CH, CH)
    thr16 = jnp.broadcast_to(jax.nn.sigmoid(s), (16,))
    t2 = _repack(table.T)
    diffs, regs = _sc_gather(u2, p2, n2, thr16, t2)
    loss, le, reg, sl = _tc_final(diffs.reshape(B // CH, CH), regs, s)
    return loss[0], le[0], reg[0], sl[0]


# speedup vs baseline: 1.3622x; 1.3622x over previous
import functools

import jax
import jax.numpy as jnp
from jax import lax
from jax.experimental import pallas as pl
from jax.experimental.pallas import tpu as pltpu
from jax.experimental.pallas import tpu_sc as plsc

B = 16384
D = 64
RB = 4096
NRB = 245
V2 = NRB * (RB // 2)
NC = 2
NS = 16
NW = NC * NS
BPW = B // NW
NCH = 4
CH = BPW // NCH
NPASS = 2
CPP = NCH // NPASS
PCH = BPW // NPASS
REG_W = 1e-4
SREG_W = 1e-3

_mesh = plsc.VectorSubcoreMesh(
    core_axis_name="c", subcore_axis_name="s", num_cores=NC, num_subcores=NS
)


@functools.partial(
    pl.kernel,
    out_type=[
        jax.ShapeDtypeStruct((B,), jnp.float32),
        jax.ShapeDtypeStruct((NW, 16), jnp.float32),
    ],
    mesh=_mesh,
    scratch_types=[
        pltpu.VMEM((NCH, CH), jnp.int32),
        pltpu.VMEM((NCH, CH), jnp.int32),
        pltpu.VMEM((NCH, CH), jnp.int32),
        pltpu.VMEM((NCH, CH), jnp.int32),
        pltpu.VMEM((NCH, CH), jnp.int32),
        pltpu.VMEM((NCH, CH), jnp.int32),
        pltpu.VMEM((PCH, 2 * D), jnp.float32),
        pltpu.VMEM((PCH, 2 * D), jnp.float32),
        pltpu.VMEM((PCH, 2 * D), jnp.float32),
        pltpu.VMEM((BPW,), jnp.float32),
        pltpu.VMEM((16,), jnp.float32),
        pltpu.VMEM((16,), jnp.float32),
        pltpu.SemaphoreType.DMA,
    ],
)
def _sc_gather(u_idx, p_idx, n_idx, thr, t2, diffs_out, regs_out,
               iu, ip_, in_, hu, hp, hn, ru, rp, rn, dbuf, thrv, rstage, sem):
    wid = lax.axis_index("s") * NC + lax.axis_index("c")
    ibase = wid * NCH

    pltpu.sync_copy(u_idx.at[pl.ds(ibase, NCH)], iu)
    pltpu.sync_copy(p_idx.at[pl.ds(ibase, NCH)], ip_)
    pltpu.sync_copy(n_idx.at[pl.ds(ibase, NCH)], in_)
    pltpu.sync_copy(thr, thrv)

    def pack_ids(j, c):
        for src, dst in ((iu, hu), (ip_, hp), (in_, hn)):
            def hbody(g, cc):
                sl = pl.ds(g * 16, 16)
                v = src[j, sl]
                dst[j, sl] = (
                    lax.shift_left(lax.shift_right_logical(v, 12), 11)
                    | (v & 2047)
                )
                return cc
            lax.fori_loop(0, CH // 16, hbody, 0)
        return c

    for j in range(NCH):
        pack_ids(j, 0)

    t = thrv[...]
    nt = -t
    lanes = lax.iota(jnp.int32, 16)

    def lane_sum(x):
        for sft in (8, 4, 2, 1):
            x = x + x.at[lanes ^ sft].get(mode="promise_in_bounds")
        return x

    racc = jnp.zeros((16,), jnp.float32)
    for ps in range(NPASS):
        copies = []
        for jj in range(CPP):
            j = ps * CPP + jj
            dst = pl.ds(jj * CH, CH)
            copies.append(pltpu.async_copy(t2.at[hu.at[j]], ru.at[dst], sem))
            copies.append(pltpu.async_copy(t2.at[hp.at[j]], rp.at[dst], sem))
            copies.append(pltpu.async_copy(t2.at[hn.at[j]], rn.at[dst], sem))
        for c in copies:
            c.wait()

        for jj in range(CPP):
            j = ps * CPP + jj

            def group(g, racc, jj=jj, j=j):
                isl = pl.ds(g * 16, 16)
                ou = iu[j, isl]
                op = ip_[j, isl]
                on = in_[j, isl]
                dv = jnp.zeros((16,), jnp.float32)
                for r in range(16):
                    row = jj * CH + g * 16 + r
                    fu = (lax.shift_right_logical(ou[r], 11) & 1) * D
                    fp = (lax.shift_right_logical(op[r], 11) & 1) * D
                    fn = (lax.shift_right_logical(on[r], 11) & 1) * D
                    acc = jnp.zeros((16,), jnp.float32)
                    for k in range(D // 16):
                        u = ru[row, pl.ds(fu + k * 16, 16)]
                        p = rp[row, pl.ds(fp + k * 16, 16)]
                        n = rn[row, pl.ds(fn + k * 16, 16)]
                        su = u - jnp.minimum(jnp.maximum(u, nt), t)
                        sp = p - jnp.minimum(jnp.maximum(p, nt), t)
                        sn = n - jnp.minimum(jnp.maximum(n, nt), t)
                        acc = acc + su * (sn - sp)
                        racc = racc + (u * u + p * p + n * n)
                    dv = jnp.where(lanes == r, lane_sum(acc), dv)
                dbuf[pl.ds(j * CH + g * 16, 16)] = dv
                return racc

            racc = lax.fori_loop(0, CH // 16, group, racc)

    rstage[...] = racc
    pltpu.sync_copy(dbuf, diffs_out.at[pl.ds(wid * BPW, BPW)])
    pltpu.sync_copy(rstage, regs_out.at[wid])


def _repack_body(x_ref, o_ref):
    x = x_ref[...]
    i0 = lax.broadcasted_iota(jnp.int32, (D, D), 0)
    i1 = lax.broadcasted_iota(jnp.int32, (D, D), 1)
    ident = (i0 == i1).astype(jnp.float32)
    xt = lax.dot_general(x, ident, (((0,), (0,)), ((), ())),
                         preferred_element_type=jnp.float32)
    o_ref[:, 0:D] = xt[0:RB // 2]
    o_ref[:, D:2 * D] = xt[RB // 2:RB]


_repack = pl.pallas_call(
    _repack_body,
    grid=(NRB,),
    in_specs=[pl.BlockSpec((D, RB), lambda b: (0, b))],
    out_specs=pl.BlockSpec((RB // 2, 2 * D), lambda b: (b, 0)),
    out_shape=jax.ShapeDtypeStruct((V2, 2 * D), jnp.float32),
    compiler_params=pltpu.CompilerParams(fuse_transposed_lhs_in_matmul=True),
)


def _tc_body(d_ref, r_ref, s_ref, loss_ref, le_ref, reg_ref, sl_ref):
    diff = d_ref[...]
    le = jnp.mean(jax.nn.softplus(diff))
    reg = 0.5 * jnp.sum(r_ref[...]) * (1.0 / B) * REG_W
    sv = s_ref[0]
    sl = 0.5 * sv * sv * (1.0 / B) * SREG_W
    le_ref[0] = le
    reg_ref[0] = reg
    sl_ref[0] = sl
    loss_ref[0] = le + reg + sl


_tc_final = pl.pallas_call(
    _tc_body,
    out_shape=[jax.ShapeDtypeStruct((1,), jnp.float32)] * 4,
    in_specs=[
        pl.BlockSpec(memory_space=pltpu.VMEM),
        pl.BlockSpec(memory_space=pltpu.VMEM),
        pl.BlockSpec(memory_space=pltpu.SMEM),
    ],
    out_specs=[pl.BlockSpec(memory_space=pltpu.SMEM)] * 4,
)


def kernel(users, pos, neg, table, s):
    s = s.astype(jnp.float32)
    u2 = users.astype(jnp.int32).reshape(B // CH, CH)
    p2 = pos.astype(jnp.int32).reshape(B // CH, CH)
    n2 = neg.astype(jnp.int32).reshape(B // CH, CH)
    thr16 = jnp.broadcast_to(jax.nn.sigmoid(s), (16,))
    t2 = _repack(table.T)
    diffs, regs = _sc_gather(u2, p2, n2, thr16, t2)
    loss, le, reg, sl = _tc_final(diffs.reshape(B // CH, CH), regs, s)
    return loss[0], le[0], reg[0], sl[0]


# speedup vs baseline: 1.6712x; 1.2269x over previous
import functools

import jax
import jax.numpy as jnp
from jax import lax
from jax.experimental import pallas as pl
from jax.experimental.pallas import tpu as pltpu
from jax.experimental.pallas import tpu_sc as plsc

B = 16384
D = 64
RB = 8192
NRB = 123
V2 = NRB * (RB // 2)
NC = 2
NS = 16
NW = NC * NS
BPW = B // NW
NCH = 4
CH = BPW // NCH
NPASS = 2
CPP = NCH // NPASS
PCH = BPW // NPASS
REG_W = 1e-4
SREG_W = 1e-3

_mesh = plsc.VectorSubcoreMesh(
    core_axis_name="c", subcore_axis_name="s", num_cores=NC, num_subcores=NS
)


@functools.partial(
    pl.kernel,
    out_type=[
        jax.ShapeDtypeStruct((B,), jnp.float32),
        jax.ShapeDtypeStruct((NW, 16), jnp.float32),
    ],
    mesh=_mesh,
    scratch_types=[
        pltpu.VMEM((NCH, CH), jnp.int32),
        pltpu.VMEM((NCH, CH), jnp.int32),
        pltpu.VMEM((NCH, CH), jnp.int32),
        pltpu.VMEM((NCH, CH), jnp.int32),
        pltpu.VMEM((NCH, CH), jnp.int32),
        pltpu.VMEM((NCH, CH), jnp.int32),
        pltpu.VMEM((PCH, 2 * D), jnp.float32),
        pltpu.VMEM((PCH, 2 * D), jnp.float32),
        pltpu.VMEM((PCH, 2 * D), jnp.float32),
        pltpu.VMEM((BPW,), jnp.float32),
        pltpu.VMEM((16,), jnp.float32),
        pltpu.VMEM((16,), jnp.float32),
        pltpu.SemaphoreType.DMA,
    ],
)
def _sc_gather(u_idx, p_idx, n_idx, thr, t2, diffs_out, regs_out,
               iu, ip_, in_, hu, hp, hn, ru, rp, rn, dbuf, thrv, rstage, sem):
    wid = lax.axis_index("s") * NC + lax.axis_index("c")
    ibase = wid * NCH

    pltpu.sync_copy(u_idx.at[pl.ds(ibase, NCH)], iu)
    pltpu.sync_copy(p_idx.at[pl.ds(ibase, NCH)], ip_)
    pltpu.sync_copy(n_idx.at[pl.ds(ibase, NCH)], in_)
    pltpu.sync_copy(thr, thrv)

    def pack_ids(j, c):
        for src, dst in ((iu, hu), (ip_, hp), (in_, hn)):
            def hbody(g, cc):
                sl = pl.ds(g * 16, 16)
                v = src[j, sl]
                dst[j, sl] = (
                    lax.shift_left(lax.shift_right_logical(v, 13), 12)
                    | (v & 4095)
                )
                return cc
            lax.fori_loop(0, CH // 16, hbody, 0)
        return c

    for j in range(NCH):
        pack_ids(j, 0)

    t = thrv[...]
    nt = -t
    lanes = lax.iota(jnp.int32, 16)

    def lane_sum(x):
        for sft in (8, 4, 2, 1):
            x = x + x.at[lanes ^ sft].get(mode="promise_in_bounds")
        return x

    racc = jnp.zeros((16,), jnp.float32)
    for ps in range(NPASS):
        copies = []
        for jj in range(CPP):
            j = ps * CPP + jj
            dst = pl.ds(jj * CH, CH)
            copies.append(pltpu.async_copy(t2.at[hu.at[j]], ru.at[dst], sem))
            copies.append(pltpu.async_copy(t2.at[hp.at[j]], rp.at[dst], sem))
            copies.append(pltpu.async_copy(t2.at[hn.at[j]], rn.at[dst], sem))
        for c in copies:
            c.wait()

        for jj in range(CPP):
            j = ps * CPP + jj

            def group(g, racc, jj=jj, j=j):
                isl = pl.ds(g * 16, 16)
                ou = iu[j, isl]
                op = ip_[j, isl]
                on = in_[j, isl]
                dv = jnp.zeros((16,), jnp.float32)
                for r in range(16):
                    row = jj * CH + g * 16 + r
                    fu = (lax.shift_right_logical(ou[r], 12) & 1) * D
                    fp = (lax.shift_right_logical(op[r], 12) & 1) * D
                    fn = (lax.shift_right_logical(on[r], 12) & 1) * D
                    acc = jnp.zeros((16,), jnp.float32)
                    for k in range(D // 16):
                        u = ru[row, pl.ds(fu + k * 16, 16)]
                        p = rp[row, pl.ds(fp + k * 16, 16)]
                        n = rn[row, pl.ds(fn + k * 16, 16)]
                        su = u - jnp.minimum(jnp.maximum(u, nt), t)
                        sp = p - jnp.minimum(jnp.maximum(p, nt), t)
                        sn = n - jnp.minimum(jnp.maximum(n, nt), t)
                        acc = acc + su * (sn - sp)
                        racc = racc + (u * u + p * p + n * n)
                    dv = jnp.where(lanes == r, lane_sum(acc), dv)
                dbuf[pl.ds(j * CH + g * 16, 16)] = dv
                return racc

            racc = lax.fori_loop(0, CH // 16, group, racc)

    rstage[...] = racc
    pltpu.sync_copy(dbuf, diffs_out.at[pl.ds(wid * BPW, BPW)])
    pltpu.sync_copy(rstage, regs_out.at[wid])


def _repack_body(x_ref, o_ref):
    x = x_ref[...]
    i0 = lax.broadcasted_iota(jnp.int32, (D, D), 0)
    i1 = lax.broadcasted_iota(jnp.int32, (D, D), 1)
    ident = (i0 == i1).astype(jnp.float32)
    xt = lax.dot_general(x, ident, (((0,), (0,)), ((), ())),
                         preferred_element_type=jnp.float32)
    o_ref[:, 0:D] = xt[0:RB // 2]
    o_ref[:, D:2 * D] = xt[RB // 2:RB]


_repack = pl.pallas_call(
    _repack_body,
    grid=(NRB,),
    in_specs=[pl.BlockSpec((D, RB), lambda b: (0, b))],
    out_specs=pl.BlockSpec((RB // 2, 2 * D), lambda b: (b, 0)),
    out_shape=jax.ShapeDtypeStruct((V2, 2 * D), jnp.float32),
    compiler_params=pltpu.CompilerParams(fuse_transposed_lhs_in_matmul=True),
)


def _tc_body(d_ref, r_ref, s_ref, loss_ref, le_ref, reg_ref, sl_ref):
    diff = d_ref[...]
    le = jnp.mean(jax.nn.softplus(diff))
    reg = 0.5 * jnp.sum(r_ref[...]) * (1.0 / B) * REG_W
    sv = s_ref[0]
    sl = 0.5 * sv * sv * (1.0 / B) * SREG_W
    le_ref[0] = le
    reg_ref[0] = reg
    sl_ref[0] = sl
    loss_ref[0] = le + reg + sl


_tc_final = pl.pallas_call(
    _tc_body,
    out_shape=[jax.ShapeDtypeStruct((1,), jnp.float32)] * 4,
    in_specs=[
        pl.BlockSpec(memory_space=pltpu.VMEM),
        pl.BlockSpec(memory_space=pltpu.VMEM),
        pl.BlockSpec(memory_space=pltpu.SMEM),
    ],
    out_specs=[pl.BlockSpec(memory_space=pltpu.SMEM)] * 4,
)


def kernel(users, pos, neg, table, s):
    s = s.astype(jnp.float32)
    u2 = users.astype(jnp.int32).reshape(B // CH, CH)
    p2 = pos.astype(jnp.int32).reshape(B // CH, CH)
    n2 = neg.astype(jnp.int32).reshape(B // CH, CH)
    thr16 = jnp.broadcast_to(jax.nn.sigmoid(s), (16,))
    t2 = _repack(table.T)
    diffs, regs = _sc_gather(u2, p2, n2, thr16, t2)
    loss, le, reg, sl = _tc_final(diffs.reshape(B // CH, CH), regs, s)
    return loss[0], le[0], reg[0], sl[0]


# speedup vs baseline: 1.8736x; 1.1211x over previous
import functools

import jax
import jax.numpy as jnp
from jax import lax
from jax.experimental import pallas as pl
from jax.experimental.pallas import tpu as pltpu
from jax.experimental.pallas import tpu_sc as plsc

B = 16384
D = 64
RB = 16384
NRB = 62
V2 = NRB * (RB // 2)
NC = 2
NS = 16
NW = NC * NS
BPW = B // NW
NCH = 4
CH = BPW // NCH
NPASS = 2
CPP = NCH // NPASS
PCH = BPW // NPASS
REG_W = 1e-4
SREG_W = 1e-3

_mesh = plsc.VectorSubcoreMesh(
    core_axis_name="c", subcore_axis_name="s", num_cores=NC, num_subcores=NS
)


@functools.partial(
    pl.kernel,
    out_type=[
        jax.ShapeDtypeStruct((B,), jnp.float32),
        jax.ShapeDtypeStruct((NW, 16), jnp.float32),
    ],
    mesh=_mesh,
    scratch_types=[
        pltpu.VMEM((NCH, CH), jnp.int32),
        pltpu.VMEM((NCH, CH), jnp.int32),
        pltpu.VMEM((NCH, CH), jnp.int32),
        pltpu.VMEM((NCH, CH), jnp.int32),
        pltpu.VMEM((NCH, CH), jnp.int32),
        pltpu.VMEM((NCH, CH), jnp.int32),
        pltpu.VMEM((PCH, 2 * D), jnp.float32),
        pltpu.VMEM((PCH, 2 * D), jnp.float32),
        pltpu.VMEM((PCH, 2 * D), jnp.float32),
        pltpu.VMEM((BPW,), jnp.float32),
        pltpu.VMEM((16,), jnp.float32),
        pltpu.VMEM((16,), jnp.float32),
        pltpu.SemaphoreType.DMA,
    ],
)
def _sc_gather(u_idx, p_idx, n_idx, thr, t2, diffs_out, regs_out,
               iu, ip_, in_, hu, hp, hn, ru, rp, rn, dbuf, thrv, rstage, sem):
    wid = lax.axis_index("s") * NC + lax.axis_index("c")
    ibase = wid * NCH

    pltpu.sync_copy(u_idx.at[pl.ds(ibase, NCH)], iu)
    pltpu.sync_copy(p_idx.at[pl.ds(ibase, NCH)], ip_)
    pltpu.sync_copy(n_idx.at[pl.ds(ibase, NCH)], in_)
    pltpu.sync_copy(thr, thrv)

    def pack_ids(j, c):
        for src, dst in ((iu, hu), (ip_, hp), (in_, hn)):
            def hbody(g, cc):
                sl = pl.ds(g * 16, 16)
                v = src[j, sl]
                dst[j, sl] = (
                    lax.shift_left(lax.shift_right_logical(v, 14), 13)
                    | (v & 8191)
                )
                return cc
            lax.fori_loop(0, CH // 16, hbody, 0)
        return c

    for j in range(NCH):
        pack_ids(j, 0)

    t = thrv[...]
    nt = -t
    lanes = lax.iota(jnp.int32, 16)

    def lane_sum(x):
        for sft in (8, 4, 2, 1):
            x = x + x.at[lanes ^ sft].get(mode="promise_in_bounds")
        return x

    racc = jnp.zeros((16,), jnp.float32)
    for ps in range(NPASS):
        copies = []
        for jj in range(CPP):
            j = ps * CPP + jj
            dst = pl.ds(jj * CH, CH)
            copies.append(pltpu.async_copy(t2.at[hu.at[j]], ru.at[dst], sem))
            copies.append(pltpu.async_copy(t2.at[hp.at[j]], rp.at[dst], sem))
            copies.append(pltpu.async_copy(t2.at[hn.at[j]], rn.at[dst], sem))
        for c in copies:
            c.wait()

        for jj in range(CPP):
            j = ps * CPP + jj

            def group(g, racc, jj=jj, j=j):
                isl = pl.ds(g * 16, 16)
                ou = iu[j, isl]
                op = ip_[j, isl]
                on = in_[j, isl]
                dv = jnp.zeros((16,), jnp.float32)
                for r in range(16):
                    row = jj * CH + g * 16 + r
                    fu = (lax.shift_right_logical(ou[r], 13) & 1) * D
                    fp = (lax.shift_right_logical(op[r], 13) & 1) * D
                    fn = (lax.shift_right_logical(on[r], 13) & 1) * D
                    acc = jnp.zeros((16,), jnp.float32)
                    for k in range(D // 16):
                        u = ru[row, pl.ds(fu + k * 16, 16)]
                        p = rp[row, pl.ds(fp + k * 16, 16)]
                        n = rn[row, pl.ds(fn + k * 16, 16)]
                        su = u - jnp.minimum(jnp.maximum(u, nt), t)
                        sp = p - jnp.minimum(jnp.maximum(p, nt), t)
                        sn = n - jnp.minimum(jnp.maximum(n, nt), t)
                        acc = acc + su * (sn - sp)
                        racc = racc + (u * u + p * p + n * n)
                    dv = jnp.where(lanes == r, lane_sum(acc), dv)
                dbuf[pl.ds(j * CH + g * 16, 16)] = dv
                return racc

            racc = lax.fori_loop(0, CH // 16, group, racc)

    rstage[...] = racc
    pltpu.sync_copy(dbuf, diffs_out.at[pl.ds(wid * BPW, BPW)])
    pltpu.sync_copy(rstage, regs_out.at[wid])


def _repack_body(x_ref, o_ref):
    x = x_ref[...]
    i0 = lax.broadcasted_iota(jnp.int32, (D, D), 0)
    i1 = lax.broadcasted_iota(jnp.int32, (D, D), 1)
    ident = (i0 == i1).astype(jnp.float32)
    xt = lax.dot_general(x, ident, (((0,), (0,)), ((), ())),
                         preferred_element_type=jnp.float32)
    o_ref[:, 0:D] = xt[0:RB // 2]
    o_ref[:, D:2 * D] = xt[RB // 2:RB]


_repack = pl.pallas_call(
    _repack_body,
    grid=(NRB,),
    in_specs=[pl.BlockSpec((D, RB), lambda b: (0, b))],
    out_specs=pl.BlockSpec((RB // 2, 2 * D), lambda b: (b, 0)),
    out_shape=jax.ShapeDtypeStruct((V2, 2 * D), jnp.float32),
    compiler_params=pltpu.CompilerParams(fuse_transposed_lhs_in_matmul=True),
)


def _tc_body(d_ref, r_ref, s_ref, loss_ref, le_ref, reg_ref, sl_ref):
    diff = d_ref[...]
    le = jnp.mean(jax.nn.softplus(diff))
    reg = 0.5 * jnp.sum(r_ref[...]) * (1.0 / B) * REG_W
    sv = s_ref[0]
    sl = 0.5 * sv * sv * (1.0 / B) * SREG_W
    le_ref[0] = le
    reg_ref[0] = reg
    sl_ref[0] = sl
    loss_ref[0] = le + reg + sl


_tc_final = pl.pallas_call(
    _tc_body,
    out_shape=[jax.ShapeDtypeStruct((1,), jnp.float32)] * 4,
    in_specs=[
        pl.BlockSpec(memory_space=pltpu.VMEM),
        pl.BlockSpec(memory_space=pltpu.VMEM),
        pl.BlockSpec(memory_space=pltpu.SMEM),
    ],
    out_specs=[pl.BlockSpec(memory_space=pltpu.SMEM)] * 4,
)


def kernel(users, pos, neg, table, s):
    s = s.astype(jnp.float32)
    u2 = users.astype(jnp.int32).reshape(B // CH, CH)
    p2 = pos.astype(jnp.int32).reshape(B // CH, CH)
    n2 = neg.astype(jnp.int32).reshape(B // CH, CH)
    thr16 = jnp.broadcast_to(jax.nn.sigmoid(s), (16,))
    t2 = _repack(table.T)
    diffs, regs = _sc_gather(u2, p2, n2, thr16, t2)
    loss, le, reg, sl = _tc_final(diffs.reshape(B // CH, CH), regs, s)
    return loss[0], le[0], reg[0], sl[0]


# speedup vs baseline: 1.9798x; 1.0567x over previous
import functools

import jax
import jax.numpy as jnp
from jax import lax
from jax.experimental import pallas as pl
from jax.experimental.pallas import tpu as pltpu
from jax.experimental.pallas import tpu_sc as plsc

B = 16384
D = 64
RB = 32768
NRB = 31
V2 = NRB * (RB // 2)
NC = 2
NS = 16
NW = NC * NS
BPW = B // NW
NCH = 4
CH = BPW // NCH
NPASS = 2
CPP = NCH // NPASS
PCH = BPW // NPASS
REG_W = 1e-4
SREG_W = 1e-3

_mesh = plsc.VectorSubcoreMesh(
    core_axis_name="c", subcore_axis_name="s", num_cores=NC, num_subcores=NS
)


@functools.partial(
    pl.kernel,
    out_type=[
        jax.ShapeDtypeStruct((B,), jnp.float32),
        jax.ShapeDtypeStruct((NW, 16), jnp.float32),
    ],
    mesh=_mesh,
    scratch_types=[
        pltpu.VMEM((NCH, CH), jnp.int32),
        pltpu.VMEM((NCH, CH), jnp.int32),
        pltpu.VMEM((NCH, CH), jnp.int32),
        pltpu.VMEM((NCH, CH), jnp.int32),
        pltpu.VMEM((NCH, CH), jnp.int32),
        pltpu.VMEM((NCH, CH), jnp.int32),
        pltpu.VMEM((PCH, 2 * D), jnp.float32),
        pltpu.VMEM((PCH, 2 * D), jnp.float32),
        pltpu.VMEM((PCH, 2 * D), jnp.float32),
        pltpu.VMEM((BPW,), jnp.float32),
        pltpu.VMEM((16,), jnp.float32),
        pltpu.VMEM((16,), jnp.float32),
        pltpu.SemaphoreType.DMA,
    ],
)
def _sc_gather(u_idx, p_idx, n_idx, thr, t2, diffs_out, regs_out,
               iu, ip_, in_, hu, hp, hn, ru, rp, rn, dbuf, thrv, rstage, sem):
    wid = lax.axis_index("s") * NC + lax.axis_index("c")
    ibase = wid * NCH

    pltpu.sync_copy(u_idx.at[pl.ds(ibase, NCH)], iu)
    pltpu.sync_copy(p_idx.at[pl.ds(ibase, NCH)], ip_)
    pltpu.sync_copy(n_idx.at[pl.ds(ibase, NCH)], in_)
    pltpu.sync_copy(thr, thrv)

    def pack_ids(j, c):
        for src, dst in ((iu, hu), (ip_, hp), (in_, hn)):
            def hbody(g, cc):
                sl = pl.ds(g * 16, 16)
                v = src[j, sl]
                dst[j, sl] = (
                    lax.shift_left(lax.shift_right_logical(v, 15), 14)
                    | (v & 16383)
                )
                return cc
            lax.fori_loop(0, CH // 16, hbody, 0)
        return c

    for j in range(NCH):
        pack_ids(j, 0)

    t = thrv[...]
    nt = -t
    lanes = lax.iota(jnp.int32, 16)

    def lane_sum(x):
        for sft in (8, 4, 2, 1):
            x = x + x.at[lanes ^ sft].get(mode="promise_in_bounds")
        return x

    racc = jnp.zeros((16,), jnp.float32)
    for ps in range(NPASS):
        copies = []
        for jj in range(CPP):
            j = ps * CPP + jj
            dst = pl.ds(jj * CH, CH)
            copies.append(pltpu.async_copy(t2.at[hu.at[j]], ru.at[dst], sem))
            copies.append(pltpu.async_copy(t2.at[hp.at[j]], rp.at[dst], sem))
            copies.append(pltpu.async_copy(t2.at[hn.at[j]], rn.at[dst], sem))
        for c in copies:
            c.wait()

        for jj in range(CPP):
            j = ps * CPP + jj

            def group(g, racc, jj=jj, j=j):
                isl = pl.ds(g * 16, 16)
                ou = iu[j, isl]
                op = ip_[j, isl]
                on = in_[j, isl]
                dv = jnp.zeros((16,), jnp.float32)
                for r in range(16):
                    row = jj * CH + g * 16 + r
                    fu = (lax.shift_right_logical(ou[r], 14) & 1) * D
                    fp = (lax.shift_right_logical(op[r], 14) & 1) * D
                    fn = (lax.shift_right_logical(on[r], 14) & 1) * D
                    acc = jnp.zeros((16,), jnp.float32)
                    for k in range(D // 16):
                        u = ru[row, pl.ds(fu + k * 16, 16)]
                        p = rp[row, pl.ds(fp + k * 16, 16)]
                        n = rn[row, pl.ds(fn + k * 16, 16)]
                        su = u - jnp.minimum(jnp.maximum(u, nt), t)
                        sp = p - jnp.minimum(jnp.maximum(p, nt), t)
                        sn = n - jnp.minimum(jnp.maximum(n, nt), t)
                        acc = acc + su * (sn - sp)
                        racc = racc + (u * u + p * p + n * n)
                    dv = jnp.where(lanes == r, lane_sum(acc), dv)
                dbuf[pl.ds(j * CH + g * 16, 16)] = dv
                return racc

            racc = lax.fori_loop(0, CH // 16, group, racc)

    rstage[...] = racc
    pltpu.sync_copy(dbuf, diffs_out.at[pl.ds(wid * BPW, BPW)])
    pltpu.sync_copy(rstage, regs_out.at[wid])


def _repack_body(x_ref, o_ref):
    x = x_ref[...]
    i0 = lax.broadcasted_iota(jnp.int32, (D, D), 0)
    i1 = lax.broadcasted_iota(jnp.int32, (D, D), 1)
    ident = (i0 == i1).astype(jnp.float32)
    xt = lax.dot_general(x, ident, (((0,), (0,)), ((), ())),
                         preferred_element_type=jnp.float32)
    o_ref[:, 0:D] = xt[0:RB // 2]
    o_ref[:, D:2 * D] = xt[RB // 2:RB]


_repack = pl.pallas_call(
    _repack_body,
    grid=(NRB,),
    in_specs=[pl.BlockSpec((D, RB), lambda b: (0, b))],
    out_specs=pl.BlockSpec((RB // 2, 2 * D), lambda b: (b, 0)),
    out_shape=jax.ShapeDtypeStruct((V2, 2 * D), jnp.float32),
    compiler_params=pltpu.CompilerParams(fuse_transposed_lhs_in_matmul=True),
)


def _tc_body(d_ref, r_ref, s_ref, loss_ref, le_ref, reg_ref, sl_ref):
    diff = d_ref[...]
    le = jnp.mean(jax.nn.softplus(diff))
    reg = 0.5 * jnp.sum(r_ref[...]) * (1.0 / B) * REG_W
    sv = s_ref[0]
    sl = 0.5 * sv * sv * (1.0 / B) * SREG_W
    le_ref[0] = le
    reg_ref[0] = reg
    sl_ref[0] = sl
    loss_ref[0] = le + reg + sl


_tc_final = pl.pallas_call(
    _tc_body,
    out_shape=[jax.ShapeDtypeStruct((1,), jnp.float32)] * 4,
    in_specs=[
        pl.BlockSpec(memory_space=pltpu.VMEM),
        pl.BlockSpec(memory_space=pltpu.VMEM),
        pl.BlockSpec(memory_space=pltpu.SMEM),
    ],
    out_specs=[pl.BlockSpec(memory_space=pltpu.SMEM)] * 4,
)


def kernel(users, pos, neg, table, s):
    s = s.astype(jnp.float32)
    u2 = users.astype(jnp.int32).reshape(B // CH, CH)
    p2 = pos.astype(jnp.int32).reshape(B // CH, CH)
    n2 = neg.astype(jnp.int32).reshape(B // CH, CH)
    thr16 = jnp.broadcast_to(jax.nn.sigmoid(s), (16,))
    t2 = _repack(table.T)
    diffs, regs = _sc_gather(u2, p2, n2, thr16, t2)
    loss, le, reg, sl = _tc_final(diffs.reshape(B // CH, CH), regs, s)
    return loss[0], le[0], reg[0], sl[0]
